# baseline (speedup 1.0000x reference)
import functools

import jax
import jax.numpy as jnp
from jax import lax
from jax.experimental import pallas as pl
from jax.experimental.pallas import tpu as pltpu
from jax.experimental.pallas import tpu_sc as plsc

E = 8
S = 2048
M = 1024
F = 4096

TS = 256
FB = 1024


LW = 128


def _gating_body(gates_ref, pos_ref, g16_ref, off_ref, laux_ref):
    gates = gates_ref[...]

    gm = jnp.max(gates, axis=-1, keepdims=True)
    eq = (gates == gm).astype(jnp.float32)
    ri = lax.broadcasted_iota(jnp.int32, (LW, LW), 0)
    ci = lax.broadcasted_iota(jnp.int32, (LW, LW), 1)
    inc_pref = (ri <= ci).astype(jnp.float32)
    cum = jnp.dot(eq, inc_pref, preferred_element_type=jnp.float32)
    mask = eq * (cum == 1.0).astype(jnp.float32)

    me = jnp.mean(gates, axis=0, keepdims=True)
    ce = jnp.mean(mask, axis=0, keepdims=True)
    laux_ref[...] = jnp.sum(me * ce, keepdims=True) * E

    gate_val = jnp.sum(gates * mask, axis=-1, keepdims=True)
    g16_ref[...] = jnp.broadcast_to(gate_val, (S, 128))

    counts = jnp.sum(mask, axis=0, keepdims=True)
    excl_pref = (ri < ci).astype(jnp.float32)
    off_excl = jnp.dot(counts, excl_pref,
                       precision=jax.lax.Precision.HIGHEST,
                       preferred_element_type=jnp.float32)

    BS = 256
    rr = lax.broadcasted_iota(jnp.int32, (BS, BS), 0)
    cc = lax.broadcasted_iota(jnp.int32, (BS, BS), 1)
    lts = (cc < rr).astype(jnp.float32)
    base = jnp.zeros((1, LW), jnp.float32)
    ranks = []
    for b in range(S // BS):
        mb = mask[b * BS:(b + 1) * BS, :]
        intra = jnp.dot(lts, mb, preferred_element_type=jnp.float32)
        ranks.append(intra + base)
        base = base + jnp.sum(mb, axis=0, keepdims=True)
    rank = jnp.concatenate(ranks, axis=0)

    pos = jnp.sum(mask * (rank + off_excl), axis=-1, keepdims=True)
    pos_ref[...] = pos.astype(jnp.int32)

    off_full = jnp.concatenate(
        [off_excl[:, :E], jnp.full((1, E), float(S), jnp.float32)], axis=1)
    off_ref[...] = off_full.astype(jnp.int32)


def _gating_call(gates_pad):
    return pl.pallas_call(
        _gating_body,
        out_shape=(
            jax.ShapeDtypeStruct((S, 1), jnp.int32),
            jax.ShapeDtypeStruct((S, 128), jnp.float32),
            jax.ShapeDtypeStruct((1, 2 * E), jnp.int32),
            jax.ShapeDtypeStruct((1, 1), jnp.float32),
        ),
    )(gates_pad)


def _make_sc_calls():
    info = plsc.get_sparse_core_info()
    nc, ns = info.num_cores, info.num_subcores
    nw = nc * ns
    bpw = S // nw
    mesh = plsc.VectorSubcoreMesh(core_axis_name="c", subcore_axis_name="s")

    @functools.partial(
        pl.kernel,
        mesh=mesh,
        out_type=(
            jax.ShapeDtypeStruct((S, M), jnp.float32),
            jax.ShapeDtypeStruct((S, 128), jnp.float32),
        ),
        scratch_types=[
            pltpu.VMEM((bpw,), jnp.int32),
            pltpu.VMEM((bpw, M), jnp.float32),
            pltpu.VMEM((bpw, 128), jnp.float32),
            pltpu.SemaphoreType.DMA,
            pltpu.SemaphoreType.DMA,
        ],
    )
    def scatter_k(x_hbm, g_hbm, pos_hbm, xs_hbm, gs_hbm,
                  idx_v, rows_v, g_v, sem1, sem2):
        wid = lax.axis_index("s") * nc + lax.axis_index("c")
        base = wid * bpw
        pltpu.sync_copy(pos_hbm.at[pl.ds(base, bpw)], idx_v)
        pltpu.sync_copy(x_hbm.at[pl.ds(base, bpw)], rows_v)
        pltpu.sync_copy(g_hbm.at[pl.ds(base, bpw)], g_v)
        c1 = pltpu.async_copy(rows_v, xs_hbm.at[idx_v], sem1)
        c2 = pltpu.async_copy(g_v, gs_hbm.at[idx_v], sem2)
        c1.wait()
        c2.wait()

    @functools.partial(
        pl.kernel,
        mesh=mesh,
        out_type=jax.ShapeDtypeStruct((S, M), jnp.float32),
        scratch_types=[
            pltpu.VMEM((bpw,), jnp.int32),
            pltpu.VMEM((bpw, M), jnp.float32),
            pltpu.SemaphoreType.DMA,
        ],
    )
    def gather_k(so_hbm, pos_hbm, out_hbm, idx_v, rows_v, sem):
        wid = lax.axis_index("s") * nc + lax.axis_index("c")
        base = wid * bpw
        pltpu.sync_copy(pos_hbm.at[pl.ds(base, bpw)], idx_v)
        pltpu.async_copy(so_hbm.at[idx_v], rows_v, sem).wait()
        pltpu.sync_copy(rows_v, out_hbm.at[pl.ds(base, bpw)])

    return scatter_k, gather_k


def _ffn_body(off_ref, xs_ref, gs_ref, w1_ref, w2_ref, out_ref):
    e = pl.program_id(0)
    f = pl.program_id(1)
    t = pl.program_id(2)

    @pl.when((e == 0) & (f == 0) & (t == 0))
    def _init():
        out_ref[...] = jnp.zeros_like(out_ref)

    lo = off_ref[e]
    hi = off_ref[e + 1]
    tstart = t * TS

    @pl.when((lo < tstart + TS) & (hi > tstart))
    def _compute():
        ts0 = pl.multiple_of(t * TS, TS)
        xs = xs_ref[pl.ds(ts0, TS), :]
        g = gs_ref[pl.ds(ts0, TS), 0:1]
        rows = ts0 + lax.broadcasted_iota(jnp.int32, (TS, 1), 0)
        gm = jnp.where((rows >= lo) & (rows < hi), g, 0.0)
        h = jnp.maximum(
            jnp.dot(xs, w1_ref[0], preferred_element_type=jnp.float32), 0.0)
        o = jnp.dot(h, w2_ref[0], preferred_element_type=jnp.float32)
        out_ref[pl.ds(ts0, TS), :] += o * gm


def _ffn_call(offs, xs, gs, w1, w2):
    grid = (E, F // FB, S // TS)
    grid_spec = pltpu.PrefetchScalarGridSpec(
        num_scalar_prefetch=1,
        grid=grid,
        in_specs=[
            pl.BlockSpec((S, M), lambda e, f, t, o: (0, 0)),
            pl.BlockSpec((S, 128), lambda e, f, t, o: (0, 0)),
            pl.BlockSpec((1, M, FB), lambda e, f, t, o: (e, 0, f)),
            pl.BlockSpec((1, FB, M), lambda e, f, t, o: (e, f, 0)),
        ],
        out_specs=pl.BlockSpec((S, M), lambda e, f, t, o: (0, 0)),
    )
    return pl.pallas_call(
        _ffn_body,
        grid_spec=grid_spec,
        out_shape=jax.ShapeDtypeStruct((S, M), jnp.float32),
        compiler_params=pltpu.CompilerParams(
            dimension_semantics=("arbitrary", "arbitrary", "arbitrary")),
    )(offs, xs, gs, w1, w2)


def kernel(hidden_states, wg, w1, w2):
    gates = jax.nn.softmax(hidden_states @ wg, axis=-1)
    gates_pad = jnp.pad(gates, ((0, 0), (0, LW - E)))
    pos2, gate16, off2, laux = _gating_call(gates_pad)
    pos = pos2.reshape(S)
    offs = off2.reshape(2 * E)

    scatter_k, gather_k = _make_sc_calls()
    xs, gs = scatter_k(hidden_states, gate16, pos)
    so = _ffn_call(offs, xs, gs, w1, w2)
    out = gather_k(so, pos)
    return out, laux.reshape(())


# speedup vs baseline: 1.0939x; 1.0939x over previous
import functools

import jax
import jax.numpy as jnp
from jax import lax
from jax.experimental import pallas as pl
from jax.experimental.pallas import tpu as pltpu
from jax.experimental.pallas import tpu_sc as plsc

E = 8
S = 2048
M = 1024
F = 4096

TS = 128
FB = 1024
CT = S + E * TS
TT = CT // TS


LW = 128


def _gating_body(gates_ref, pos_ref, g16_ref, off_ref, laux_ref):
    gates = gates_ref[...]

    gm = jnp.max(gates, axis=-1, keepdims=True)
    eq = (gates == gm).astype(jnp.float32)
    ri = lax.broadcasted_iota(jnp.int32, (LW, LW), 0)
    ci = lax.broadcasted_iota(jnp.int32, (LW, LW), 1)
    inc_pref = (ri <= ci).astype(jnp.float32)
    cum = jnp.dot(eq, inc_pref, preferred_element_type=jnp.float32)
    mask = eq * (cum == 1.0).astype(jnp.float32)

    me = jnp.mean(gates, axis=0, keepdims=True)
    ce = jnp.mean(mask, axis=0, keepdims=True)
    laux_ref[...] = jnp.sum(me * ce, keepdims=True) * E

    gate_val = jnp.sum(gates * mask, axis=-1, keepdims=True)
    g16_ref[...] = jnp.broadcast_to(gate_val, (S, 128))

    counts = jnp.sum(mask, axis=0, keepdims=True)
    excl_pref = (ri < ci).astype(jnp.float32)
    pc = jnp.floor((counts + float(TS - 1)) * (1.0 / TS)) * float(TS)
    off_excl = jnp.dot(pc, excl_pref,
                       precision=jax.lax.Precision.HIGHEST,
                       preferred_element_type=jnp.float32)

    BS = 256
    rr = lax.broadcasted_iota(jnp.int32, (BS, BS), 0)
    cc = lax.broadcasted_iota(jnp.int32, (BS, BS), 1)
    lts = (cc < rr).astype(jnp.float32)
    base = jnp.zeros((1, LW), jnp.float32)
    ranks = []
    for b in range(S // BS):
        mb = mask[b * BS:(b + 1) * BS, :]
        intra = jnp.dot(lts, mb, preferred_element_type=jnp.float32)
        ranks.append(intra + base)
        base = base + jnp.sum(mb, axis=0, keepdims=True)
    rank = jnp.concatenate(ranks, axis=0)

    pos = jnp.sum(mask * (rank + off_excl), axis=-1, keepdims=True)
    pos_ref[...] = pos.astype(jnp.int32)

    off_ref[...] = off_excl[:, :2 * E].astype(jnp.int32)


def _gating_call(gates_pad):
    return pl.pallas_call(
        _gating_body,
        out_shape=(
            jax.ShapeDtypeStruct((S, 1), jnp.int32),
            jax.ShapeDtypeStruct((S, 128), jnp.float32),
            jax.ShapeDtypeStruct((1, 2 * E), jnp.int32),
            jax.ShapeDtypeStruct((1, 1), jnp.float32),
        ),
    )(gates_pad)


def _make_sc_calls():
    info = plsc.get_sparse_core_info()
    nc, ns = info.num_cores, info.num_subcores
    nw = nc * ns
    bpw = S // nw
    mesh = plsc.VectorSubcoreMesh(core_axis_name="c", subcore_axis_name="s")

    @functools.partial(
        pl.kernel,
        mesh=mesh,
        out_type=(
            jax.ShapeDtypeStruct((CT, M), jnp.float32),
            jax.ShapeDtypeStruct((CT, 128), jnp.float32),
        ),
        scratch_types=[
            pltpu.VMEM((bpw,), jnp.int32),
            pltpu.VMEM((bpw, M), jnp.float32),
            pltpu.VMEM((bpw, 128), jnp.float32),
            pltpu.SemaphoreType.DMA,
            pltpu.SemaphoreType.DMA,
        ],
    )
    def scatter_k(x_hbm, g_hbm, pos_hbm, xs_hbm, gs_hbm,
                  idx_v, rows_v, g_v, sem1, sem2):
        wid = lax.axis_index("s") * nc + lax.axis_index("c")
        base = wid * bpw
        pltpu.sync_copy(pos_hbm.at[pl.ds(base, bpw)], idx_v)
        pltpu.sync_copy(x_hbm.at[pl.ds(base, bpw)], rows_v)
        pltpu.sync_copy(g_hbm.at[pl.ds(base, bpw)], g_v)
        c1 = pltpu.async_copy(rows_v, xs_hbm.at[idx_v], sem1)
        c2 = pltpu.async_copy(g_v, gs_hbm.at[idx_v], sem2)
        c1.wait()
        c2.wait()

    @functools.partial(
        pl.kernel,
        mesh=mesh,
        out_type=jax.ShapeDtypeStruct((S, M), jnp.float32),

        scratch_types=[
            pltpu.VMEM((bpw,), jnp.int32),
            pltpu.VMEM((bpw, M), jnp.float32),
            pltpu.SemaphoreType.DMA,
        ],
    )
    def gather_k(so_hbm, pos_hbm, out_hbm, idx_v, rows_v, sem):
        wid = lax.axis_index("s") * nc + lax.axis_index("c")
        base = wid * bpw
        pltpu.sync_copy(pos_hbm.at[pl.ds(base, bpw)], idx_v)
        pltpu.async_copy(so_hbm.at[idx_v], rows_v, sem).wait()
        pltpu.sync_copy(rows_v, out_hbm.at[pl.ds(base, bpw)])

    return scatter_k, gather_k


def _ffn_body(meta_ref, xs_ref, gs_ref, w1_ref, w2_ref, out_ref):
    f = pl.program_id(0)
    t = pl.program_id(1)

    @pl.when(t < meta_ref[0])
    def _compute():
        ts0 = pl.multiple_of(t * TS, TS)
        xs = xs_ref[pl.ds(ts0, TS), :]
        g = gs_ref[pl.ds(ts0, TS), 0:1]
        h = jnp.maximum(
            jnp.dot(xs, w1_ref[0], preferred_element_type=jnp.float32), 0.0)
        o = jnp.dot(h, w2_ref[0], preferred_element_type=jnp.float32)

        @pl.when(f == 0)
        def _first():
            out_ref[pl.ds(ts0, TS), :] = o * g

        @pl.when(f > 0)
        def _acc():
            out_ref[pl.ds(ts0, TS), :] += o * g


def _ffn_call(meta, xs, gs, w1, w2):
    grid = (F // FB, TT)
    grid_spec = pltpu.PrefetchScalarGridSpec(
        num_scalar_prefetch=1,
        grid=grid,
        in_specs=[
            pl.BlockSpec((CT, M), lambda f, t, m: (0, 0)),
            pl.BlockSpec((CT, 128), lambda f, t, m: (0, 0)),
            pl.BlockSpec((1, M, FB), lambda f, t, m: (m[1 + t], 0, f)),
            pl.BlockSpec((1, FB, M), lambda f, t, m: (m[1 + t], f, 0)),
        ],
        out_specs=pl.BlockSpec((CT, M), lambda f, t, m: (0, 0)),
    )
    return pl.pallas_call(
        _ffn_body,
        grid_spec=grid_spec,
        out_shape=jax.ShapeDtypeStruct((CT, M), jnp.float32),
        compiler_params=pltpu.CompilerParams(
            dimension_semantics=("arbitrary", "arbitrary")),
    )(meta, xs, gs, w1, w2)


def kernel(hidden_states, wg, w1, w2):
    gates = jax.nn.softmax(hidden_states @ wg, axis=-1)
    gates_pad = jnp.pad(gates, ((0, 0), (0, LW - E)))
    pos2, gate16, off2, laux = _gating_call(gates_pad)
    pos = pos2.reshape(S)
    aoff = off2.reshape(2 * E)

    ntiles = aoff[E] // TS
    tstarts = jnp.arange(TT, dtype=jnp.int32) * TS
    te = jnp.sum((aoff[None, :E] <= tstarts[:, None]).astype(jnp.int32),
                 axis=1) - 1
    meta = jnp.concatenate([ntiles[None], te]).astype(jnp.int32)

    scatter_k, gather_k = _make_sc_calls()
    xs, gs = scatter_k(hidden_states, gate16, pos)
    so = _ffn_call(meta, xs, gs, w1, w2)
    out = gather_k(so, pos)
    return out, laux.reshape(())
